# Initial kernel scaffold; baseline (speedup 1.0000x reference)
import jax
import jax.numpy as jnp
from jax.experimental import pallas as pl


def kernel(x, edge_index, edge_attr, W_in, b_in, We_0, be_0, W1_0, b1_0, g_0, bt_0, W2_0, b2_0, We_1, be_1, W1_1, b1_1, g_1, bt_1, W2_1, b2_1):
    raise NotImplementedError("write your pallas kernel here")


# speedup vs baseline: 4.3719x; 4.3719x over previous
import functools

import jax
import jax.numpy as jnp
from jax import lax
from jax.experimental import pallas as pl
from jax.experimental.pallas import tpu as pltpu
from jax.experimental.pallas import tpu_sc as plsc

EMB = 64
LANES = 16
NC = 2
NS = 16
CHUNK = 400
BN = 2000


def _sc_mesh():
    return plsc.VectorSubcoreMesh(core_axis_name="c", subcore_axis_name="s")


_SC_PARAMS = pltpu.CompilerParams(use_tc_tiling_on_sc=False,
                                  needs_layout_passes=False)


def _sc_segsum_call(h, src, dst):
    n_nodes = h.shape[0]
    n_edges = src.shape[0]
    half = n_nodes // 2
    per_sub = -(-(half + 1) // (NS * CHUNK)) * CHUNK
    rpad = per_sub * NS
    trash = half
    last = half - (NS - 1) * per_sub
    epw = n_edges // NS
    nchunks = epw // CHUNK
    assert epw * NS == n_edges and nchunks * CHUNK == epw and last > 0

    def body(h_hbm, src_hbm, dst_hbm, zeros_hbm, out_hbm,
             src_v, dst_v, loc_v, rows_v, acc_sh, sem):
        c = lax.axis_index("c")
        s = lax.axis_index("s")
        for i in range(per_sub // CHUNK):
            pltpu.sync_copy(zeros_hbm,
                            acc_sh.at[pl.ds(s * per_sub + i * CHUNK, CHUNK)])
        plsc.subcore_barrier()

        base0 = s * epw
        off = c * half

        @pl.loop(0, nchunks)
        def _(k):
            base = base0 + k * CHUNK
            pltpu.sync_copy(src_hbm.at[pl.ds(base, CHUNK)], src_v)
            pltpu.sync_copy(dst_hbm.at[pl.ds(base, CHUNK)], dst_v)
            gat = pltpu.async_copy(h_hbm.at[src_v], rows_v, sem)
            for i in range(CHUNK // LANES):
                sl = pl.ds(i * LANES, LANES)
                d = dst_v[sl] - off
                ok = (d >= 0) & (d < half)
                loc_v[sl] = jnp.where(ok, d, trash)
            gat.wait()
            pltpu.sync_copy(rows_v, acc_sh.at[loc_v], add=True)

        plsc.subcore_barrier()

        @pl.when(s < NS - 1)
        def _():
            pltpu.sync_copy(acc_sh.at[pl.ds(s * per_sub, per_sub)],
                            out_hbm.at[pl.ds(c * half + s * per_sub, per_sub)])

        @pl.when(s == NS - 1)
        def _():
            pltpu.sync_copy(acc_sh.at[pl.ds((NS - 1) * per_sub, last)],
                            out_hbm.at[pl.ds(c * half + (NS - 1) * per_sub, last)])

    run = pl.kernel(
        body,
        out_type=jax.ShapeDtypeStruct((n_nodes, EMB), jnp.float32),
        mesh=_sc_mesh(),
        scratch_types=[
            pltpu.VMEM((CHUNK,), jnp.int32),
            pltpu.VMEM((CHUNK,), jnp.int32),
            pltpu.VMEM((CHUNK,), jnp.int32),
            pltpu.VMEM((CHUNK, EMB), jnp.float32),
            pltpu.VMEM_SHARED((rpad, EMB), jnp.float32),
            pltpu.SemaphoreType.DMA,
        ],
        compiler_params=_SC_PARAMS,
    )
    return run(h, src, dst, jnp.zeros((CHUNK, EMB), jnp.float32))


def _sc_scalar_call(attr, dst, n_nodes):
    n_edges = dst.shape[0]
    half = n_nodes // 2
    per_sub = -(-(half + 1) // (NS * CHUNK)) * CHUNK
    rpad = per_sub * NS
    trash = half
    last = half - (NS - 1) * per_sub
    epw = n_edges // NS
    nchunks = epw // CHUNK
    assert epw * NS == n_edges and nchunks * CHUNK == epw and last > 0

    def body(attr_hbm, dst_hbm, zeros_hbm, out_hbm,
             attr_v, dst_v, loc_v, rows_v, acc_sh, sem):
        c = lax.axis_index("c")
        s = lax.axis_index("s")
        pltpu.sync_copy(zeros_hbm, acc_sh.at[pl.ds(s * per_sub, per_sub)])
        pltpu.sync_copy(zeros_hbm.at[pl.ds(0, CHUNK)], rows_v)
        plsc.subcore_barrier()

        base0 = s * epw
        off = c * half
        ones = jnp.ones((LANES,), jnp.float32)
        col0 = jnp.zeros((LANES,), jnp.int32)
        col1 = col0 + 1

        @pl.loop(0, nchunks)
        def _(k):
            base = base0 + k * CHUNK
            pltpu.sync_copy(attr_hbm.at[pl.ds(base, CHUNK)], attr_v)
            pltpu.sync_copy(dst_hbm.at[pl.ds(base, CHUNK)], dst_v)
            for i in range(CHUNK // LANES):
                sl = pl.ds(i * LANES, LANES)
                ridx = lax.iota(jnp.int32, LANES) + (i * LANES)
                plsc.store_scatter(rows_v, [ridx, col0], attr_v[sl])
                plsc.store_scatter(rows_v, [ridx, col1], ones)
                d = dst_v[sl] - off
                ok = (d >= 0) & (d < half)
                loc_v[sl] = jnp.where(ok, d, trash)
            pltpu.sync_copy(rows_v, acc_sh.at[loc_v], add=True)

        plsc.subcore_barrier()

        @pl.when(s < NS - 1)
        def _():
            pltpu.sync_copy(acc_sh.at[pl.ds(s * per_sub, per_sub)],
                            out_hbm.at[pl.ds(c * half + s * per_sub, per_sub)])

        @pl.when(s == NS - 1)
        def _():
            pltpu.sync_copy(acc_sh.at[pl.ds((NS - 1) * per_sub, last)],
                            out_hbm.at[pl.ds(c * half + (NS - 1) * per_sub, last)])

    run = pl.kernel(
        body,
        out_type=jax.ShapeDtypeStruct((n_nodes, LANES), jnp.float32),
        mesh=_sc_mesh(),
        scratch_types=[
            pltpu.VMEM((CHUNK,), jnp.float32),
            pltpu.VMEM((CHUNK,), jnp.int32),
            pltpu.VMEM((CHUNK,), jnp.int32),
            pltpu.VMEM((CHUNK, LANES), jnp.float32),
            pltpu.VMEM_SHARED((rpad, LANES), jnp.float32),
            pltpu.SemaphoreType.DMA,
        ],
        compiler_params=_SC_PARAMS,
    )
    return run(attr, dst, jnp.zeros((per_sub, LANES), jnp.float32))


def _embed_call(x2d, W_in, b_in):
    n, f = x2d.shape

    def body(x_ref, w_ref, b_ref, o_ref):
        o_ref[...] = (jnp.dot(x_ref[...], w_ref[...],
                              preferred_element_type=jnp.float32)
                      + b_ref[0:1, :])

    return pl.pallas_call(
        body,
        grid=(n // BN,),
        in_specs=[pl.BlockSpec((BN, f), lambda i: (i, 0)),
                  pl.BlockSpec((f, EMB), lambda i: (0, 0)),
                  pl.BlockSpec((8, EMB), lambda i: (0, 0))],
        out_specs=pl.BlockSpec((BN, EMB), lambda i: (i, 0)),
        out_shape=jax.ShapeDtypeStruct((n, EMB), jnp.float32),
    )(x2d, W_in, jnp.broadcast_to(b_in.reshape(1, EMB), (8, EMB)))


def _z_block(a_ref, sd_ref, w1_ref, we_ref, be_ref, b1_ref):
    w1lo = w1_ref[0:EMB, :]
    w1hi = w1_ref[EMB:, :]
    u = jnp.dot(we_ref[0:1, :], w1hi, preferred_element_type=jnp.float32)
    w = jnp.dot(be_ref[0:1, :], w1hi, preferred_element_type=jnp.float32)
    return (jnp.dot(a_ref[...], w1lo, preferred_element_type=jnp.float32)
            + sd_ref[:, 0:1] * u + sd_ref[:, 1:2] * w + b1_ref[0:1, :])


def _gin_stats_call(aggr, sd, W1, we8, be8, b18):
    n = aggr.shape[0]
    d2 = W1.shape[1]

    def body(a_ref, sd_ref, w1_ref, we_ref, be_ref, b1_ref, sum_ref, sq_ref):
        i = pl.program_id(0)
        z = _z_block(a_ref, sd_ref, w1_ref, we_ref, be_ref, b1_ref)
        ps = jnp.sum(z, axis=0, keepdims=True)
        pq = jnp.sum(z * z, axis=0, keepdims=True)

        @pl.when(i == 0)
        def _():
            sum_ref[...] = jnp.zeros_like(sum_ref)
            sq_ref[...] = jnp.zeros_like(sq_ref)

        sum_ref[0:1, :] += ps
        sq_ref[0:1, :] += pq

    return pl.pallas_call(
        body,
        grid=(n // BN,),
        in_specs=[pl.BlockSpec((BN, EMB), lambda i: (i, 0)),
                  pl.BlockSpec((BN, LANES), lambda i: (i, 0)),
                  pl.BlockSpec((2 * EMB, d2), lambda i: (0, 0)),
                  pl.BlockSpec((8, EMB), lambda i: (0, 0)),
                  pl.BlockSpec((8, EMB), lambda i: (0, 0)),
                  pl.BlockSpec((8, d2), lambda i: (0, 0))],
        out_specs=[pl.BlockSpec((8, d2), lambda i: (0, 0)),
                   pl.BlockSpec((8, d2), lambda i: (0, 0))],
        out_shape=[jax.ShapeDtypeStruct((8, d2), jnp.float32),
                   jax.ShapeDtypeStruct((8, d2), jnp.float32)],
    )(aggr, sd, W1, we8, be8, b18)


def _gin_norm_call(aggr, sd, W1, we8, be8, b18, sums, sq, g8, bt8, W2, b28,
                   relu_out):
    n = aggr.shape[0]
    d2 = W1.shape[1]
    dout = W2.shape[1]
    inv_n = 1.0 / n

    def body(a_ref, sd_ref, w1_ref, we_ref, be_ref, b1_ref, s_ref, q_ref,
             g_ref, bt_ref, w2_ref, b2_ref, o_ref):
        z = _z_block(a_ref, sd_ref, w1_ref, we_ref, be_ref, b1_ref)
        mean = s_ref[0:1, :] * inv_n
        var = q_ref[0:1, :] * inv_n - mean * mean
        zn = (z - mean) * (g_ref[0:1, :] / jnp.sqrt(var + 1e-5)) + bt_ref[0:1, :]
        zn = jnp.maximum(zn, 0.0)
        o = (jnp.dot(zn, w2_ref[...], preferred_element_type=jnp.float32)
             + b2_ref[0:1, :])
        if relu_out:
            o = jnp.maximum(o, 0.0)
        o_ref[...] = o

    return pl.pallas_call(
        body,
        grid=(n // BN,),
        in_specs=[pl.BlockSpec((BN, EMB), lambda i: (i, 0)),
                  pl.BlockSpec((BN, LANES), lambda i: (i, 0)),
                  pl.BlockSpec((2 * EMB, d2), lambda i: (0, 0)),
                  pl.BlockSpec((8, EMB), lambda i: (0, 0)),
                  pl.BlockSpec((8, EMB), lambda i: (0, 0)),
                  pl.BlockSpec((8, d2), lambda i: (0, 0)),
                  pl.BlockSpec((8, d2), lambda i: (0, 0)),
                  pl.BlockSpec((8, d2), lambda i: (0, 0)),
                  pl.BlockSpec((8, d2), lambda i: (0, 0)),
                  pl.BlockSpec((8, d2), lambda i: (0, 0)),
                  pl.BlockSpec((d2, dout), lambda i: (0, 0)),
                  pl.BlockSpec((8, dout), lambda i: (0, 0))],
        out_specs=pl.BlockSpec((BN, dout), lambda i: (i, 0)),
        out_shape=jax.ShapeDtypeStruct((n, dout), jnp.float32),
    )(aggr, sd, W1, we8, be8, b18, sums, sq, g8, bt8, W2, b28)


def _b8(v):
    v = v.reshape(1, -1)
    return jnp.broadcast_to(v, (8, v.shape[1]))


def _gin_layer(aggr, sd, We, be, W1, b1, g, bt, W2, b2, relu_out):
    we8, be8, b18 = _b8(We), _b8(be), _b8(b1)
    sums, sq = _gin_stats_call(aggr, sd, W1, we8, be8, b18)
    return _gin_norm_call(aggr, sd, W1, we8, be8, b18, sums, sq,
                          _b8(g), _b8(bt), W2, _b8(b2), relu_out)


def kernel(x, edge_index, edge_attr, W_in, b_in, We_0, be_0, W1_0, b1_0, g_0,
           bt_0, W2_0, b2_0, We_1, be_1, W1_1, b1_1, g_1, bt_1, W2_1, b2_1):
    src = edge_index[0, 0]
    dst = edge_index[0, 1]
    attr = edge_attr[0, :, 0]
    n_nodes = x.shape[1]

    h0 = _embed_call(x[0], W_in, b_in)
    sd = _sc_scalar_call(attr, dst, n_nodes)
    aggr0 = _sc_segsum_call(h0, src, dst)
    h1 = _gin_layer(aggr0, sd, We_0, be_0, W1_0, b1_0, g_0, bt_0, W2_0, b2_0,
                    True)
    aggr1 = _sc_segsum_call(h1, src, dst)
    h2 = _gin_layer(aggr1, sd, We_1, be_1, W1_1, b1_1, g_1, bt_1, W2_1, b2_1,
                    False)
    return h2[None, :, :]


# speedup vs baseline: 4.4710x; 1.0227x over previous
import functools

import jax
import jax.numpy as jnp
from jax import lax
from jax.experimental import pallas as pl
from jax.experimental.pallas import tpu as pltpu
from jax.experimental.pallas import tpu_sc as plsc

EMB = 64
LANES = 16
NC = 2
NS = 16
CHUNK_G = 200
CHUNK_S = 2000
BN = 2000


def _sc_mesh():
    return plsc.VectorSubcoreMesh(core_axis_name="c", subcore_axis_name="s")


_SC_PARAMS = pltpu.CompilerParams(use_tc_tiling_on_sc=False,
                                  needs_layout_passes=False)


def _sc_segsum_call(h, src, dst):
    n_nodes = h.shape[0]
    n_edges = src.shape[0]
    CHUNK = 200
    SUP = 8
    half = n_nodes // 2
    per_sub = -(-(half + 1) // (NS * 8)) * 8
    rpad = per_sub * NS
    trash = half
    last = half - (NS - 1) * per_sub
    epw = n_edges // NS
    nchunks = epw // CHUNK
    assert epw * NS == n_edges and nchunks * CHUNK == epw and last > 0
    n_sup = nchunks // SUP
    tail_ch = nchunks - n_sup * SUP
    assert tail_ch % 2 == 0 and per_sub % 4 == 0
    gbases = list(range(0, CHUNK - 15, LANES))
    if gbases[-1] != CHUNK - LANES:
        gbases.append(CHUNK - LANES)

    def body(h_hbm, src_hbm, dst_hbm, zeros_hbm, out_hbm,
             srcb_v, dstb_v, loc_v, rows_v, acc_sh, gsem0, gsem1, ssem0,
             ssem1):
        gsem = [gsem0, gsem1]
        ssem = [ssem0, ssem1]
        c = lax.axis_index("c")
        s = lax.axis_index("s")
        for i in range(4):
            pltpu.sync_copy(
                zeros_hbm,
                acc_sh.at[pl.ds(s * per_sub + i * (per_sub // 4),
                                per_sub // 4)])
        plsc.subcore_barrier()

        base0 = s * epw
        off = c * half

        def transform(j, b):
            for gb in gbases:
                d = dstb_v[pl.ds(j * CHUNK + gb, LANES)] - off
                ok = (d >= 0) & (d < half)
                loc_v[b, pl.ds(gb, LANES)] = jnp.where(ok, d, trash)

        def g_issue(j, b):
            return pltpu.async_copy(
                h_hbm.at[srcb_v.at[pl.ds(j * CHUNK, CHUNK)]], rows_v.at[b],
                gsem[b])

        def s_issue(b):
            return pltpu.async_copy(rows_v.at[b], acc_sh.at[loc_v.at[b]],
                                    ssem[b], add=True)

        def do_pair(j):
            transform(j, 0)
            g0 = g_issue(j, 0)
            transform(j + 1, 1)
            g0.wait()
            s0 = s_issue(0)
            g1 = g_issue(j + 1, 1)
            s0.wait()
            g1.wait()
            s_issue(1).wait()

        @pl.loop(0, n_sup)
        def _(u):
            base = base0 + u * (SUP * CHUNK)
            pltpu.sync_copy(src_hbm.at[pl.ds(base, SUP * CHUNK)], srcb_v)
            pltpu.sync_copy(dst_hbm.at[pl.ds(base, SUP * CHUNK)], dstb_v)
            for j in range(0, SUP, 2):
                do_pair(j)

        if tail_ch:
            base = base0 + n_sup * (SUP * CHUNK)
            nt = tail_ch * CHUNK
            pltpu.sync_copy(src_hbm.at[pl.ds(base, nt)],
                            srcb_v.at[pl.ds(0, nt)])
            pltpu.sync_copy(dst_hbm.at[pl.ds(base, nt)],
                            dstb_v.at[pl.ds(0, nt)])
            for j in range(0, tail_ch, 2):
                do_pair(j)

        plsc.subcore_barrier()

        @pl.when(s < NS - 1)
        def _():
            pltpu.sync_copy(acc_sh.at[pl.ds(s * per_sub, per_sub)],
                            out_hbm.at[pl.ds(c * half + s * per_sub, per_sub)])

        @pl.when(s == NS - 1)
        def _():
            pltpu.sync_copy(acc_sh.at[pl.ds((NS - 1) * per_sub, last)],
                            out_hbm.at[pl.ds(c * half + (NS - 1) * per_sub, last)])

    run = pl.kernel(
        body,
        out_type=jax.ShapeDtypeStruct((n_nodes, EMB), jnp.float32),
        mesh=_sc_mesh(),
        scratch_types=[
            pltpu.VMEM((SUP * CHUNK,), jnp.int32),
            pltpu.VMEM((SUP * CHUNK,), jnp.int32),
            pltpu.VMEM((2, CHUNK), jnp.int32),
            pltpu.VMEM((2, CHUNK, EMB), jnp.float32),
            pltpu.VMEM_SHARED((rpad, EMB), jnp.float32),
            pltpu.SemaphoreType.DMA,
            pltpu.SemaphoreType.DMA,
            pltpu.SemaphoreType.DMA,
            pltpu.SemaphoreType.DMA,
        ],
        compiler_params=_SC_PARAMS,
    )
    return run(h, src, dst, jnp.zeros((per_sub // 4, EMB), jnp.float32))


def _sc_scalar_call(attr, dst, n_nodes):
    n_edges = dst.shape[0]
    half = n_nodes // 2
    per_sub = -(-(half + 1) // (NS * 8)) * 8
    rpad = per_sub * NS
    trash = half
    last = half - (NS - 1) * per_sub
    epw = n_edges // NS
    nchunks = epw // CHUNK_S
    assert epw * NS == n_edges and nchunks * CHUNK_S == epw and last > 0
    assert per_sub <= CHUNK_S

    NB = 2
    n_loop = nchunks // NB
    tail = nchunks - n_loop * NB
    CHUNK = CHUNK_S

    def body(attr_hbm, dst_hbm, zeros_hbm, out_hbm,
             attr_v, loc_v, rows_v, acc_sh, ssem0, ssem1):
        ssem = [ssem0, ssem1]
        c = lax.axis_index("c")
        s = lax.axis_index("s")
        ones = jnp.ones((LANES,), jnp.float32)
        col0 = jnp.zeros((LANES,), jnp.int32)
        col1 = col0 + 1
        pltpu.sync_copy(zeros_hbm.at[pl.ds(0, per_sub)],
                        acc_sh.at[pl.ds(s * per_sub, per_sub)])
        for b in range(NB):
            pltpu.sync_copy(zeros_hbm.at[pl.ds(0, CHUNK)], rows_v.at[b])
            for i in range(CHUNK // LANES):
                ridx = lax.iota(jnp.int32, LANES) + (i * LANES)
                plsc.store_scatter(rows_v.at[b], [ridx, col1], ones)
        plsc.subcore_barrier()

        base0 = s * epw
        off = c * half

        def load_transform(m, b):
            base = base0 + m * CHUNK
            pltpu.sync_copy(attr_hbm.at[pl.ds(base, CHUNK)], attr_v.at[b])
            pltpu.sync_copy(dst_hbm.at[pl.ds(base, CHUNK)], loc_v.at[b])
            for i in range(CHUNK // LANES):
                sl = (b, pl.ds(i * LANES, LANES))
                ridx = lax.iota(jnp.int32, LANES) + (i * LANES)
                plsc.store_scatter(rows_v.at[b], [ridx, col0], attr_v[sl])
                d = loc_v[sl] - off
                ok = (d >= 0) & (d < half)
                loc_v[sl] = jnp.where(ok, d, trash)

        def s_issue(b):
            return pltpu.async_copy(rows_v.at[b], acc_sh.at[loc_v.at[b]],
                                    ssem[b], add=True)

        @pl.loop(0, n_loop)
        def _(g):
            load_transform(g * NB, 0)
            sd0 = s_issue(0)
            load_transform(g * NB + 1, 1)
            sd0.wait()
            s_issue(1).wait()

        for t in range(tail):
            b = t % NB
            load_transform(n_loop * NB + t, b)
            s_issue(b).wait()

        plsc.subcore_barrier()

        @pl.when(s < NS - 1)
        def _():
            pltpu.sync_copy(acc_sh.at[pl.ds(s * per_sub, per_sub)],
                            out_hbm.at[pl.ds(c * half + s * per_sub, per_sub)])

        @pl.when(s == NS - 1)
        def _():
            pltpu.sync_copy(acc_sh.at[pl.ds((NS - 1) * per_sub, last)],
                            out_hbm.at[pl.ds(c * half + (NS - 1) * per_sub, last)])

    run = pl.kernel(
        body,
        out_type=jax.ShapeDtypeStruct((n_nodes, LANES), jnp.float32),
        mesh=_sc_mesh(),
        scratch_types=[
            pltpu.VMEM((NB, CHUNK), jnp.float32),
            pltpu.VMEM((NB, CHUNK), jnp.int32),
            pltpu.VMEM((NB, CHUNK, LANES), jnp.float32),
            pltpu.VMEM_SHARED((rpad, LANES), jnp.float32),
            pltpu.SemaphoreType.DMA,
            pltpu.SemaphoreType.DMA,
        ],
        compiler_params=_SC_PARAMS,
    )
    return run(attr, dst, jnp.zeros((max(per_sub, CHUNK_S), LANES),
                                    jnp.float32))


def _embed_call(x2d, W_in, b_in):
    n, f = x2d.shape

    def body(x_ref, w_ref, b_ref, o_ref):
        o_ref[...] = (jnp.dot(x_ref[...], w_ref[...],
                              preferred_element_type=jnp.float32)
                      + b_ref[0:1, :])

    return pl.pallas_call(
        body,
        grid=(n // BN,),
        in_specs=[pl.BlockSpec((BN, f), lambda i: (i, 0)),
                  pl.BlockSpec((f, EMB), lambda i: (0, 0)),
                  pl.BlockSpec((8, EMB), lambda i: (0, 0))],
        out_specs=pl.BlockSpec((BN, EMB), lambda i: (i, 0)),
        out_shape=jax.ShapeDtypeStruct((n, EMB), jnp.float32),
    )(x2d, W_in, jnp.broadcast_to(b_in.reshape(1, EMB), (8, EMB)))


def _z_block(a_ref, sd_ref, w1_ref, we_ref, be_ref, b1_ref):
    w1lo = w1_ref[0:EMB, :]
    w1hi = w1_ref[EMB:, :]
    u = jnp.dot(we_ref[0:1, :], w1hi, preferred_element_type=jnp.float32)
    w = jnp.dot(be_ref[0:1, :], w1hi, preferred_element_type=jnp.float32)
    return (jnp.dot(a_ref[...], w1lo, preferred_element_type=jnp.float32)
            + sd_ref[:, 0:1] * u + sd_ref[:, 1:2] * w + b1_ref[0:1, :])


def _gin_stats_call(aggr, sd, W1, we8, be8, b18):
    n = aggr.shape[0]
    d2 = W1.shape[1]

    def body(a_ref, sd_ref, w1_ref, we_ref, be_ref, b1_ref, sum_ref, sq_ref):
        i = pl.program_id(0)
        z = _z_block(a_ref, sd_ref, w1_ref, we_ref, be_ref, b1_ref)
        ps = jnp.sum(z, axis=0, keepdims=True)
        pq = jnp.sum(z * z, axis=0, keepdims=True)

        @pl.when(i == 0)
        def _():
            sum_ref[...] = jnp.zeros_like(sum_ref)
            sq_ref[...] = jnp.zeros_like(sq_ref)

        sum_ref[0:1, :] += ps
        sq_ref[0:1, :] += pq

    return pl.pallas_call(
        body,
        grid=(n // BN,),
        in_specs=[pl.BlockSpec((BN, EMB), lambda i: (i, 0)),
                  pl.BlockSpec((BN, LANES), lambda i: (i, 0)),
                  pl.BlockSpec((2 * EMB, d2), lambda i: (0, 0)),
                  pl.BlockSpec((8, EMB), lambda i: (0, 0)),
                  pl.BlockSpec((8, EMB), lambda i: (0, 0)),
                  pl.BlockSpec((8, d2), lambda i: (0, 0))],
        out_specs=[pl.BlockSpec((8, d2), lambda i: (0, 0)),
                   pl.BlockSpec((8, d2), lambda i: (0, 0))],
        out_shape=[jax.ShapeDtypeStruct((8, d2), jnp.float32),
                   jax.ShapeDtypeStruct((8, d2), jnp.float32)],
    )(aggr, sd, W1, we8, be8, b18)


def _gin_norm_call(aggr, sd, W1, we8, be8, b18, sums, sq, g8, bt8, W2, b28,
                   relu_out):
    n = aggr.shape[0]
    d2 = W1.shape[1]
    dout = W2.shape[1]
    inv_n = 1.0 / n

    def body(a_ref, sd_ref, w1_ref, we_ref, be_ref, b1_ref, s_ref, q_ref,
             g_ref, bt_ref, w2_ref, b2_ref, o_ref):
        z = _z_block(a_ref, sd_ref, w1_ref, we_ref, be_ref, b1_ref)
        mean = s_ref[0:1, :] * inv_n
        var = q_ref[0:1, :] * inv_n - mean * mean
        zn = (z - mean) * (g_ref[0:1, :] / jnp.sqrt(var + 1e-5)) + bt_ref[0:1, :]
        zn = jnp.maximum(zn, 0.0)
        o = (jnp.dot(zn, w2_ref[...], preferred_element_type=jnp.float32)
             + b2_ref[0:1, :])
        if relu_out:
            o = jnp.maximum(o, 0.0)
        o_ref[...] = o

    return pl.pallas_call(
        body,
        grid=(n // BN,),
        in_specs=[pl.BlockSpec((BN, EMB), lambda i: (i, 0)),
                  pl.BlockSpec((BN, LANES), lambda i: (i, 0)),
                  pl.BlockSpec((2 * EMB, d2), lambda i: (0, 0)),
                  pl.BlockSpec((8, EMB), lambda i: (0, 0)),
                  pl.BlockSpec((8, EMB), lambda i: (0, 0)),
                  pl.BlockSpec((8, d2), lambda i: (0, 0)),
                  pl.BlockSpec((8, d2), lambda i: (0, 0)),
                  pl.BlockSpec((8, d2), lambda i: (0, 0)),
                  pl.BlockSpec((8, d2), lambda i: (0, 0)),
                  pl.BlockSpec((8, d2), lambda i: (0, 0)),
                  pl.BlockSpec((d2, dout), lambda i: (0, 0)),
                  pl.BlockSpec((8, dout), lambda i: (0, 0))],
        out_specs=pl.BlockSpec((BN, dout), lambda i: (i, 0)),
        out_shape=jax.ShapeDtypeStruct((n, dout), jnp.float32),
    )(aggr, sd, W1, we8, be8, b18, sums, sq, g8, bt8, W2, b28)


def _b8(v):
    v = v.reshape(1, -1)
    return jnp.broadcast_to(v, (8, v.shape[1]))


def _gin_layer(aggr, sd, We, be, W1, b1, g, bt, W2, b2, relu_out):
    we8, be8, b18 = _b8(We), _b8(be), _b8(b1)
    sums, sq = _gin_stats_call(aggr, sd, W1, we8, be8, b18)
    return _gin_norm_call(aggr, sd, W1, we8, be8, b18, sums, sq,
                          _b8(g), _b8(bt), W2, _b8(b2), relu_out)


def kernel(x, edge_index, edge_attr, W_in, b_in, We_0, be_0, W1_0, b1_0, g_0,
           bt_0, W2_0, b2_0, We_1, be_1, W1_1, b1_1, g_1, bt_1, W2_1, b2_1):
    src = edge_index[0, 0]
    dst = edge_index[0, 1]
    attr = edge_attr[0, :, 0]
    n_nodes = x.shape[1]

    h0 = _embed_call(x[0], W_in, b_in)
    sd = _sc_scalar_call(attr, dst, n_nodes)
    aggr0 = _sc_segsum_call(h0, src, dst)
    h1 = _gin_layer(aggr0, sd, We_0, be_0, W1_0, b1_0, g_0, bt_0, W2_0, b2_0,
                    True)
    aggr1 = _sc_segsum_call(h1, src, dst)
    h2 = _gin_layer(aggr1, sd, We_1, be_1, W1_1, b1_1, g_1, bt_1, W2_1, b2_1,
                    False)
    return h2[None, :, :]
